# Initial kernel scaffold; baseline (speedup 1.0000x reference)
import jax
import jax.numpy as jnp
from jax.experimental import pallas as pl


def kernel(data, params):
    raise NotImplementedError("write your pallas kernel here")


# speedup vs baseline: 2429.6542x; 2429.6542x over previous
import jax
import jax.numpy as jnp
from jax.experimental import pallas as pl

L = 512
ENC = 256
DEC = 256
H = 2
B = 8
ALPHA = 0.2
F32 = jnp.float32
BF16 = jnp.bfloat16
HIGH = jax.lax.Precision.HIGHEST


def _lrelu(x):
    return jnp.where(x >= 0, x, ALPHA * x)


def _elu(x):
    return jnp.where(x > 0, x, jnp.exp(jnp.minimum(x, 0.0)) - 1.0)


def _t(x):
    return x.astype(BF16).astype(F32)


def _vgnn_kernel(data_ref, dataT_ref, embed_ref, Wenc_ref, aenc_ref,
                 Wdec_ref, adec_ref, genc_ref, benc_ref, gdec_ref, bdec_ref,
                 Vw_ref, Vb_ref, w1_ref, b1_ref, w2_ref, b2_ref, out_ref):
    embed16 = embed_ref[...].astype(BF16)

    Hs = []
    Ps = []
    for h in range(H):
        Hh = jnp.dot(embed16, Wenc_ref[h].astype(BF16),
                     preferred_element_type=F32)
        Ht = _t(Hh)
        a1 = _t(aenc_ref[h:h + 1, :ENC])
        a2 = _t(aenc_ref[h:h + 1, ENC:])
        s1 = jnp.sum(Ht * a1, axis=1, keepdims=True)
        s2 = jnp.sum(Ht * a2, axis=1, keepdims=True)
        x = s1 + s2.reshape(1, L)
        Ps.append(jnp.exp(-_lrelu(x)))
        Hs.append(Hh)

    genc = genc_ref[...]
    benc = benc_ref[...]
    enc_virt = _elu(benc)

    rows = []
    for b in range(B):
        mcol = dataT_ref[:, b:b + 1]
        total = jnp.sum(mcol)
        iota = jax.lax.broadcasted_iota(jnp.int32, (L, 1), 0)
        oh0 = (iota == 0).astype(F32)
        empty = (total == 0).astype(F32)
        me = oh0 * empty + mcol * (1.0 - empty)

        parts = []
        for h in range(H):
            Hm = Hs[h] * me
            numer = jnp.dot(Ps[h], Hm, preferred_element_type=F32,
                            precision=HIGH)
            denom = jnp.dot(Ps[h], me, preferred_element_type=F32,
                            precision=HIGH)
            hp = (me * numer) / (me * denom + 1e-10)
            parts.append(_elu(hp))
        enc_raw = jnp.concatenate(parts, axis=1)

        mu = jnp.mean(enc_raw, axis=1, keepdims=True)
        xc = enc_raw - mu
        sd = jnp.sqrt(jnp.sum(xc * xc, axis=1, keepdims=True) / (2 * ENC - 1))
        encoded = _elu(genc * xc / (sd + 1e-6) + benc)
        enc16 = encoded.astype(BF16)
        ev16 = enc_virt.astype(BF16)

        dec = jnp.zeros((1, DEC), dtype=F32)
        for h in range(H):
            Wd16 = Wdec_ref[h].astype(BF16)
            Hd = jnp.dot(enc16, Wd16, preferred_element_type=F32)
            Hv = jnp.dot(ev16, Wd16, preferred_element_type=F32)
            Hdt = _t(Hd)
            Hvt = _t(Hv)
            a1d = _t(adec_ref[h:h + 1, :DEC])
            a2d = _t(adec_ref[h:h + 1, DEC:])
            s1v = jnp.sum(Hvt * a1d)
            s2 = jnp.sum(Hdt * a2d, axis=1, keepdims=True)
            s2v = jnp.sum(Hvt * a2d)
            e = jnp.exp(-_lrelu(s1v + s2)) * mcol
            ev = jnp.exp(-_lrelu(s1v + s2v))
            den = jnp.sum(e) + ev + 1e-10
            wsum = jnp.sum(e * Hd, axis=0, keepdims=True) + ev * Hv
            dec = dec + 0.5 * wsum / den

        mu = jnp.mean(dec, axis=1, keepdims=True)
        xc = dec - mu
        sd = jnp.sqrt(jnp.sum(xc * xc, axis=1, keepdims=True) / (DEC - 1))
        lnd = gdec_ref[...] * xc / (sd + 1e-6) + bdec_ref[...]
        r = jnp.maximum(lnd, 0.0)
        rows.append(jnp.dot(r.astype(BF16), Vw_ref[...].astype(BF16),
                            preferred_element_type=F32) + Vb_ref[...])

    stacked = jnp.concatenate(rows, axis=0)
    h1 = jnp.maximum(
        jnp.dot(stacked.astype(BF16), w1_ref[...].astype(BF16),
                preferred_element_type=F32) + b1_ref[...], 0.0)
    pred = jnp.dot(h1.astype(BF16), w2_ref[...].astype(BF16),
                   preferred_element_type=F32) + b2_ref[...]
    out_ref[...] = jnp.broadcast_to(pred, (B, 128))


def kernel(data, params):
    embed512 = params["embed"][:L]
    args = (
        data.astype(F32),
        data.T.astype(F32),
        embed512,
        params["W_enc"],
        params["a_enc"],
        params["W_dec"],
        params["a_dec"],
        params["g_enc"].reshape(1, 2 * ENC),
        params["b_enc"].reshape(1, 2 * ENC),
        params["g_dec"].reshape(1, DEC),
        params["b_dec"].reshape(1, DEC),
        params["V_w"],
        params["V_b"].reshape(1, DEC),
        params["out_w1"],
        params["out_b1"].reshape(1, DEC),
        params["out_w2"],
        params["out_b2"].reshape(1, 1),
    )
    out = pl.pallas_call(
        _vgnn_kernel,
        out_shape=jax.ShapeDtypeStruct((B, 128), F32),
    )(*args)
    return (out[:, :1], jnp.asarray(0.0, dtype=F32))


# speedup vs baseline: 3672.1747x; 1.5114x over previous
import jax
import jax.numpy as jnp
from jax.experimental import pallas as pl

L = 512
ENC = 256
DEC = 256
H = 2
B = 8
ALPHA = 0.2
F32 = jnp.float32
BF16 = jnp.bfloat16
HIGH = jax.lax.Precision.HIGHEST


def _lrelu(x):
    return jnp.where(x >= 0, x, ALPHA * x)


def _elu(x):
    return jnp.where(x > 0, x, jnp.exp(jnp.minimum(x, 0.0)) - 1.0)


def _t(x):
    return x.astype(BF16).astype(F32)


def _vgnn_kernel(data_ref, dataT_ref, embed_ref, Wenc_ref, aenc_ref,
                 Wdec_ref, adec_ref, genc_ref, benc_ref, gdec_ref, bdec_ref,
                 Vw_ref, Vb_ref, w1_ref, b1_ref, w2_ref, b2_ref, out_ref):
    embed16 = embed_ref[...].astype(BF16)

    Hs = []
    Ps = []
    for h in range(H):
        Hh = jnp.dot(embed16, Wenc_ref[h].astype(BF16),
                     preferred_element_type=F32)
        Ht = _t(Hh)
        a1 = _t(aenc_ref[h:h + 1, :ENC])
        a2 = _t(aenc_ref[h:h + 1, ENC:])
        s1 = jnp.sum(Ht * a1, axis=1, keepdims=True)
        s2 = jnp.sum(Ht * a2, axis=1, keepdims=True)
        x = s1 + s2.reshape(1, L)
        Ps.append(jnp.exp(-_lrelu(x)))
        Hs.append(Hh)

    masks = dataT_ref[...]
    totals = jnp.sum(masks, axis=0, keepdims=True)
    empty = (totals == 0).astype(F32)
    iota = jax.lax.broadcasted_iota(jnp.int32, (L, 1), 0)
    oh0 = (iota == 0).astype(F32)
    me_all = oh0 * empty + masks * (1.0 - empty)

    Rs = []
    for h in range(H):
        X = jnp.concatenate(
            [Hs[h] * me_all[:, b:b + 1] for b in range(B)] + [me_all],
            axis=1)
        Rs.append(jnp.dot(Ps[h], X, preferred_element_type=F32,
                          precision=HIGH))

    blocks = []
    for b in range(B):
        me = me_all[:, b:b + 1]
        parts = []
        for h in range(H):
            numer = Rs[h][:, b * ENC:(b + 1) * ENC]
            denom = Rs[h][:, B * ENC + b:B * ENC + b + 1]
            w = me / (me * denom + 1e-10)
            parts.append(_elu(numer * w))
        blocks.append(jnp.concatenate(parts, axis=1))
    enc_raw = jnp.concatenate(blocks, axis=0)

    genc = genc_ref[...]
    benc = benc_ref[...]
    mu = jnp.mean(enc_raw, axis=1, keepdims=True)
    xc = enc_raw - mu
    sd = jnp.sqrt(jnp.sum(xc * xc, axis=1, keepdims=True) / (2 * ENC - 1))
    encoded = _elu(genc * (xc / (sd + 1e-6)) + benc)
    enc16 = encoded.astype(BF16)
    enc_virt = _elu(benc)
    ev16 = enc_virt.astype(BF16)

    Wd_all = jnp.concatenate([Wdec_ref[0], Wdec_ref[1]],
                             axis=1).astype(BF16)
    Hd_all = jnp.dot(enc16, Wd_all, preferred_element_type=F32)
    Hv_all = jnp.dot(ev16, Wd_all, preferred_element_type=F32)
    Hdt_all = _t(Hd_all)
    Hvt_all = _t(Hv_all)

    rows = []
    for b in range(B):
        mcol = dataT_ref[:, b:b + 1]
        dec = jnp.zeros((1, DEC), dtype=F32)
        for h in range(H):
            Hd = Hd_all[b * L:(b + 1) * L, h * DEC:(h + 1) * DEC]
            Hdt = Hdt_all[b * L:(b + 1) * L, h * DEC:(h + 1) * DEC]
            Hv = Hv_all[:, h * DEC:(h + 1) * DEC]
            Hvt = Hvt_all[:, h * DEC:(h + 1) * DEC]
            a1d = _t(adec_ref[h:h + 1, :DEC])
            a2d = _t(adec_ref[h:h + 1, DEC:])
            s1v = jnp.sum(Hvt * a1d)
            s2 = jnp.sum(Hdt * a2d, axis=1, keepdims=True)
            s2v = jnp.sum(Hvt * a2d)
            e = jnp.exp(-_lrelu(s1v + s2)) * mcol
            ev = jnp.exp(-_lrelu(s1v + s2v))
            den = jnp.sum(e) + ev + 1e-10
            wsum = jnp.sum(e * Hd, axis=0, keepdims=True) + ev * Hv
            dec = dec + 0.5 * wsum / den

        mu = jnp.mean(dec, axis=1, keepdims=True)
        xc = dec - mu
        sd = jnp.sqrt(jnp.sum(xc * xc, axis=1, keepdims=True) / (DEC - 1))
        lnd = gdec_ref[...] * xc / (sd + 1e-6) + bdec_ref[...]
        r = jnp.maximum(lnd, 0.0)
        rows.append(jnp.dot(r.astype(BF16), Vw_ref[...].astype(BF16),
                            preferred_element_type=F32) + Vb_ref[...])

    stacked = jnp.concatenate(rows, axis=0)
    h1 = jnp.maximum(
        jnp.dot(stacked.astype(BF16), w1_ref[...].astype(BF16),
                preferred_element_type=F32) + b1_ref[...], 0.0)
    pred = jnp.dot(h1.astype(BF16), w2_ref[...].astype(BF16),
                   preferred_element_type=F32) + b2_ref[...]
    out_ref[...] = jnp.broadcast_to(pred, (B, 128))


def kernel(data, params):
    embed512 = params["embed"][:L]
    args = (
        data.astype(F32),
        data.T.astype(F32),
        embed512,
        params["W_enc"],
        params["a_enc"],
        params["W_dec"],
        params["a_dec"],
        params["g_enc"].reshape(1, 2 * ENC),
        params["b_enc"].reshape(1, 2 * ENC),
        params["g_dec"].reshape(1, DEC),
        params["b_dec"].reshape(1, DEC),
        params["V_w"],
        params["V_b"].reshape(1, DEC),
        params["out_w1"],
        params["out_b1"].reshape(1, DEC),
        params["out_w2"],
        params["out_b2"].reshape(1, 1),
    )
    out = pl.pallas_call(
        _vgnn_kernel,
        out_shape=jax.ShapeDtypeStruct((B, 128), F32),
    )(*args)
    return (out[:, :1], jnp.asarray(0.0, dtype=F32))


# speedup vs baseline: 4331.1747x; 1.1795x over previous
import jax
import jax.numpy as jnp
from jax.experimental import pallas as pl

L = 512
ENC = 256
DEC = 256
H = 2
B = 8
ALPHA = 0.2
F32 = jnp.float32
BF16 = jnp.bfloat16
HIGH = jax.lax.Precision.HIGHEST


def _lrelu(x):
    return jnp.where(x >= 0, x, ALPHA * x)


def _elu(x):
    return jnp.where(x > 0, x, jnp.exp(jnp.minimum(x, 0.0)) - 1.0)


def _t(x):
    return x.astype(BF16).astype(F32)


def _dot3(a, x):
    ahi = a.astype(BF16)
    alo = (a - ahi.astype(F32)).astype(BF16)
    xhi = x.astype(BF16)
    xlo = (x - xhi.astype(F32)).astype(BF16)
    return (jnp.dot(ahi, xhi, preferred_element_type=F32)
            + jnp.dot(ahi, xlo, preferred_element_type=F32)
            + jnp.dot(alo, xhi, preferred_element_type=F32))


def _vgnn_kernel(data_ref, dataT_ref, embed_ref, Wenc_ref, aenc_ref,
                 Wdec_ref, adec_ref, genc_ref, benc_ref, gdec_ref, bdec_ref,
                 Vw_ref, Vb_ref, w1_ref, b1_ref, w2_ref, b2_ref, out_ref):
    embed16 = embed_ref[...].astype(BF16)

    Hs = []
    Ps = []
    for h in range(H):
        Hh = jnp.dot(embed16, Wenc_ref[h].astype(BF16),
                     preferred_element_type=F32)
        Ht = _t(Hh)
        a1 = _t(aenc_ref[h:h + 1, :ENC])
        a2 = _t(aenc_ref[h:h + 1, ENC:])
        s1 = jnp.sum(Ht * a1, axis=1, keepdims=True)
        s2 = jnp.sum(Ht * a2, axis=1, keepdims=True)
        x = s1 + s2.reshape(1, L)
        Ps.append(jnp.exp(-_lrelu(x)))
        Hs.append(Hh)

    masks = dataT_ref[...]
    totals = jnp.sum(masks, axis=0, keepdims=True)
    empty = (totals == 0).astype(F32)
    iota = jax.lax.broadcasted_iota(jnp.int32, (L, 1), 0)
    oh0 = (iota == 0).astype(F32)
    me_all = oh0 * empty + masks * (1.0 - empty)

    Rs = []
    for h in range(H):
        X = jnp.concatenate(
            [Hs[h] * me_all[:, b:b + 1] for b in range(B)] + [me_all],
            axis=1)
        Rs.append(_dot3(Ps[h], X))

    blocks = []
    for b in range(B):
        me = me_all[:, b:b + 1]
        parts = []
        for h in range(H):
            numer = Rs[h][:, b * ENC:(b + 1) * ENC]
            denom = Rs[h][:, B * ENC + b:B * ENC + b + 1]
            w = me / (me * denom + 1e-10)
            parts.append(_elu(numer * w))
        blocks.append(jnp.concatenate(parts, axis=1))
    enc_raw = jnp.concatenate(blocks, axis=0)

    genc = genc_ref[...]
    benc = benc_ref[...]
    mu = jnp.mean(enc_raw, axis=1, keepdims=True)
    xc = enc_raw - mu
    sd = jnp.sqrt(jnp.sum(xc * xc, axis=1, keepdims=True) / (2 * ENC - 1))
    encoded = _elu(genc * (xc * (1.0 / (sd + 1e-6))) + benc)
    enc16 = encoded.astype(BF16)
    enc_virt = _elu(benc)
    ev16 = enc_virt.astype(BF16)

    Wd_all = jnp.concatenate([Wdec_ref[0], Wdec_ref[1]],
                             axis=1).astype(BF16)
    Hd_all = jnp.dot(enc16, Wd_all, preferred_element_type=F32)
    Hv_all = jnp.dot(ev16, Wd_all, preferred_element_type=F32)
    Hdt_all = _t(Hd_all)
    Hvt_all = _t(Hv_all)

    rows = []
    for b in range(B):
        mcol = dataT_ref[:, b:b + 1]
        dec = jnp.zeros((1, DEC), dtype=F32)
        for h in range(H):
            Hd = Hd_all[b * L:(b + 1) * L, h * DEC:(h + 1) * DEC]
            Hdt = Hdt_all[b * L:(b + 1) * L, h * DEC:(h + 1) * DEC]
            Hv = Hv_all[:, h * DEC:(h + 1) * DEC]
            Hvt = Hvt_all[:, h * DEC:(h + 1) * DEC]
            a1d = _t(adec_ref[h:h + 1, :DEC])
            a2d = _t(adec_ref[h:h + 1, DEC:])
            s1v = jnp.sum(Hvt * a1d)
            s2 = jnp.sum(Hdt * a2d, axis=1, keepdims=True)
            s2v = jnp.sum(Hvt * a2d)
            e = jnp.exp(-_lrelu(s1v + s2)) * mcol
            ev = jnp.exp(-_lrelu(s1v + s2v))
            den = jnp.sum(e) + ev + 1e-10
            wsum = jnp.sum(e * Hd, axis=0, keepdims=True) + ev * Hv
            dec = dec + 0.5 * wsum / den

        mu = jnp.mean(dec, axis=1, keepdims=True)
        xc = dec - mu
        sd = jnp.sqrt(jnp.sum(xc * xc, axis=1, keepdims=True) / (DEC - 1))
        lnd = gdec_ref[...] * xc / (sd + 1e-6) + bdec_ref[...]
        r = jnp.maximum(lnd, 0.0)
        rows.append(jnp.dot(r.astype(BF16), Vw_ref[...].astype(BF16),
                            preferred_element_type=F32) + Vb_ref[...])

    stacked = jnp.concatenate(rows, axis=0)
    h1 = jnp.maximum(
        jnp.dot(stacked.astype(BF16), w1_ref[...].astype(BF16),
                preferred_element_type=F32) + b1_ref[...], 0.0)
    pred = jnp.dot(h1.astype(BF16), w2_ref[...].astype(BF16),
                   preferred_element_type=F32) + b2_ref[...]
    out_ref[...] = jnp.broadcast_to(pred, (B, 128))


def kernel(data, params):
    embed512 = params["embed"][:L]
    args = (
        data.astype(F32),
        data.T.astype(F32),
        embed512,
        params["W_enc"],
        params["a_enc"],
        params["W_dec"],
        params["a_dec"],
        params["g_enc"].reshape(1, 2 * ENC),
        params["b_enc"].reshape(1, 2 * ENC),
        params["g_dec"].reshape(1, DEC),
        params["b_dec"].reshape(1, DEC),
        params["V_w"],
        params["V_b"].reshape(1, DEC),
        params["out_w1"],
        params["out_b1"].reshape(1, DEC),
        params["out_w2"],
        params["out_b2"].reshape(1, 1),
    )
    out = pl.pallas_call(
        _vgnn_kernel,
        out_shape=jax.ShapeDtypeStruct((B, 128), F32),
    )(*args)
    return (out[:, :1], jnp.asarray(0.0, dtype=F32))


# speedup vs baseline: 4350.3335x; 1.0044x over previous
import jax
import jax.numpy as jnp
from jax.experimental import pallas as pl

L = 512
ENC = 256
DEC = 256
H = 2
B = 8
ALPHA = 0.2
F32 = jnp.float32
BF16 = jnp.bfloat16
HIGH = jax.lax.Precision.HIGHEST


def _lrelu(x):
    return jnp.where(x >= 0, x, ALPHA * x)


def _elu(x):
    return jnp.where(x > 0, x, jnp.exp(jnp.minimum(x, 0.0)) - 1.0)


def _t(x):
    return x.astype(BF16).astype(F32)


def _split(a):
    ahi = a.astype(BF16)
    alo = (a - ahi.astype(F32)).astype(BF16)
    return ahi, alo


def _vgnn_kernel(data_ref, dataT_ref, embed_ref, Wenc_ref, aenc_ref,
                 Wdec_ref, adec_ref, genc_ref, benc_ref, gdec_ref, bdec_ref,
                 Vw_ref, Vb_ref, w1_ref, b1_ref, w2_ref, b2_ref, out_ref):
    embed16 = embed_ref[...].astype(BF16)

    Hs = []
    Ps = []
    for h in range(H):
        Hh = jnp.dot(embed16, Wenc_ref[h].astype(BF16),
                     preferred_element_type=F32)
        Ht = _t(Hh)
        a1 = _t(aenc_ref[h:h + 1, :ENC])
        a2 = _t(aenc_ref[h:h + 1, ENC:])
        s1 = jnp.sum(Ht * a1, axis=1, keepdims=True)
        s2 = jnp.sum(Ht * a2, axis=1, keepdims=True)
        x = s1 + s2.reshape(1, L)
        Ps.append(jnp.exp(-_lrelu(x)))
        Hs.append(Hh)

    masks = dataT_ref[...]
    totals = jnp.sum(masks, axis=0, keepdims=True)
    empty = (totals == 0).astype(F32)
    iota = jax.lax.broadcasted_iota(jnp.int32, (L, 1), 0)
    oh0 = (iota == 0).astype(F32)
    me_all = oh0 * empty + masks * (1.0 - empty)

    me16 = me_all.astype(BF16)
    zeros8 = jnp.zeros((L, B), dtype=BF16)
    Rs = []
    for h in range(H):
        Phi, Plo = _split(Ps[h])
        Hhi, Hlo = _split(Hs[h])
        Xhi = jnp.concatenate(
            [Hhi * me16[:, b:b + 1] for b in range(B)] + [me16],
            axis=1)
        Xlo = jnp.concatenate(
            [Hlo * me16[:, b:b + 1] for b in range(B)] + [zeros8],
            axis=1)
        Rs.append(jnp.dot(Phi, Xhi, preferred_element_type=F32)
                  + jnp.dot(Phi, Xlo, preferred_element_type=F32)
                  + jnp.dot(Plo, Xhi, preferred_element_type=F32))

    blocks = []
    for b in range(B):
        me = me_all[:, b:b + 1]
        parts = []
        for h in range(H):
            numer = Rs[h][:, b * ENC:(b + 1) * ENC]
            denom = Rs[h][:, B * ENC + b:B * ENC + b + 1]
            w = me / (me * denom + 1e-10)
            parts.append(_elu(numer * w))
        blocks.append(jnp.concatenate(parts, axis=1))
    enc_raw = jnp.concatenate(blocks, axis=0)

    genc = genc_ref[...]
    benc = benc_ref[...]
    mu = jnp.mean(enc_raw, axis=1, keepdims=True)
    xc = enc_raw - mu
    sd = jnp.sqrt(jnp.sum(xc * xc, axis=1, keepdims=True) / (2 * ENC - 1))
    encoded = _elu(genc * (xc * (1.0 / (sd + 1e-6))) + benc)
    enc16 = encoded.astype(BF16)
    enc_virt = _elu(benc)
    ev16 = enc_virt.astype(BF16)

    Wd_all = jnp.concatenate([Wdec_ref[0], Wdec_ref[1]],
                             axis=1).astype(BF16)
    Hd_all = jnp.dot(enc16, Wd_all, preferred_element_type=F32)
    Hv_all = jnp.dot(ev16, Wd_all, preferred_element_type=F32)
    Hdt_all = _t(Hd_all)
    Hvt_all = _t(Hv_all)

    rows = []
    for b in range(B):
        mcol = dataT_ref[:, b:b + 1]
        dec = jnp.zeros((1, DEC), dtype=F32)
        for h in range(H):
            Hd = Hd_all[b * L:(b + 1) * L, h * DEC:(h + 1) * DEC]
            Hdt = Hdt_all[b * L:(b + 1) * L, h * DEC:(h + 1) * DEC]
            Hv = Hv_all[:, h * DEC:(h + 1) * DEC]
            Hvt = Hvt_all[:, h * DEC:(h + 1) * DEC]
            a1d = _t(adec_ref[h:h + 1, :DEC])
            a2d = _t(adec_ref[h:h + 1, DEC:])
            s1v = jnp.sum(Hvt * a1d)
            s2 = jnp.sum(Hdt * a2d, axis=1, keepdims=True)
            s2v = jnp.sum(Hvt * a2d)
            e = jnp.exp(-_lrelu(s1v + s2)) * mcol
            ev = jnp.exp(-_lrelu(s1v + s2v))
            den = jnp.sum(e) + ev + 1e-10
            wsum = jnp.sum(e * Hd, axis=0, keepdims=True) + ev * Hv
            dec = dec + 0.5 * wsum / den

        mu = jnp.mean(dec, axis=1, keepdims=True)
        xc = dec - mu
        sd = jnp.sqrt(jnp.sum(xc * xc, axis=1, keepdims=True) / (DEC - 1))
        lnd = gdec_ref[...] * xc / (sd + 1e-6) + bdec_ref[...]
        r = jnp.maximum(lnd, 0.0)
        rows.append(jnp.dot(r.astype(BF16), Vw_ref[...].astype(BF16),
                            preferred_element_type=F32) + Vb_ref[...])

    stacked = jnp.concatenate(rows, axis=0)
    h1 = jnp.maximum(
        jnp.dot(stacked.astype(BF16), w1_ref[...].astype(BF16),
                preferred_element_type=F32) + b1_ref[...], 0.0)
    pred = jnp.dot(h1.astype(BF16), w2_ref[...].astype(BF16),
                   preferred_element_type=F32) + b2_ref[...]
    out_ref[...] = jnp.broadcast_to(pred, (B, 128))


def kernel(data, params):
    embed512 = params["embed"][:L]
    args = (
        data.astype(F32),
        data.T.astype(F32),
        embed512,
        params["W_enc"],
        params["a_enc"],
        params["W_dec"],
        params["a_dec"],
        params["g_enc"].reshape(1, 2 * ENC),
        params["b_enc"].reshape(1, 2 * ENC),
        params["g_dec"].reshape(1, DEC),
        params["b_dec"].reshape(1, DEC),
        params["V_w"],
        params["V_b"].reshape(1, DEC),
        params["out_w1"],
        params["out_b1"].reshape(1, DEC),
        params["out_w2"],
        params["out_b2"].reshape(1, 1),
    )
    out = pl.pallas_call(
        _vgnn_kernel,
        out_shape=jax.ShapeDtypeStruct((B, 128), F32),
    )(*args)
    return (out[:, :1], jnp.asarray(0.0, dtype=F32))


# speedup vs baseline: 4966.2902x; 1.1416x over previous
import jax
import jax.numpy as jnp
from jax.experimental import pallas as pl

L = 512
ENC = 256
DEC = 256
H = 2
B = 8
ALPHA = 0.2
F32 = jnp.float32
BF16 = jnp.bfloat16
HIGH = jax.lax.Precision.HIGHEST


def _lrelu(x):
    return jnp.maximum(x, ALPHA * x)


def _elu(x):
    return jnp.maximum(x, jnp.exp(jnp.minimum(x, 0.0)) - 1.0)


def _t(x):
    return x.astype(BF16).astype(F32)


def _split(a):
    ahi = a.astype(BF16)
    alo = (a - ahi.astype(F32)).astype(BF16)
    return ahi, alo


def _vgnn_kernel(data_ref, embed_ref, Wenc_ref, aenc_ref,
                 Wdec_ref, adec_ref, genc_ref, benc_ref, gdec_ref, bdec_ref,
                 Vw_ref, Vb_ref, w1_ref, b1_ref, w2_ref, b2_ref, out_ref):
    embed16 = embed_ref[:L, :].astype(BF16)
    masksT = data_ref[...].T

    Hs = []
    Ps = []
    for h in range(H):
        Hh = jnp.dot(embed16, Wenc_ref[h].astype(BF16),
                     preferred_element_type=F32)
        Ht = _t(Hh)
        a1 = _t(aenc_ref[h:h + 1, :ENC])
        a2 = _t(aenc_ref[h:h + 1, ENC:])
        s1 = jnp.sum(Ht * a1, axis=1, keepdims=True)
        s2 = jnp.sum(Ht * a2, axis=1, keepdims=True)
        x = s1 + s2.reshape(1, L)
        Ps.append(jnp.exp(-_lrelu(x)))
        Hs.append(Hh)

    masks = masksT
    totals = jnp.sum(masks, axis=0, keepdims=True)
    empty = (totals == 0).astype(F32)
    iota = jax.lax.broadcasted_iota(jnp.int32, (L, 1), 0)
    oh0 = (iota == 0).astype(F32)
    me_all = oh0 * empty + masks * (1.0 - empty)

    me16 = me_all.astype(BF16)
    zeros8 = jnp.zeros((L, B), dtype=BF16)
    Rs = []
    for h in range(H):
        Phi, Plo = _split(Ps[h])
        Hhi, Hlo = _split(Hs[h])
        Xhi = jnp.concatenate(
            [Hhi * me16[:, b:b + 1] for b in range(B)] + [me16],
            axis=1)
        Xlo = jnp.concatenate(
            [Hlo * me16[:, b:b + 1] for b in range(B)] + [zeros8],
            axis=1)
        Rs.append(jnp.dot(Phi, Xhi, preferred_element_type=F32)
                  + jnp.dot(Phi, Xlo, preferred_element_type=F32)
                  + jnp.dot(Plo, Xhi, preferred_element_type=F32))

    blocks = []
    for b in range(B):
        me = me_all[:, b:b + 1]
        parts = []
        for h in range(H):
            numer = Rs[h][:, b * ENC:(b + 1) * ENC]
            denom = Rs[h][:, B * ENC + b:B * ENC + b + 1]
            w = me / (me * denom + 1e-10)
            parts.append(_elu(numer * w))
        blocks.append(jnp.concatenate(parts, axis=1))
    enc_raw = jnp.concatenate(blocks, axis=0)

    genc = genc_ref[...]
    benc = benc_ref[...]
    mu = jnp.mean(enc_raw, axis=1, keepdims=True)
    xc = enc_raw - mu
    sd = jnp.sqrt(jnp.sum(xc * xc, axis=1, keepdims=True) / (2 * ENC - 1))
    encoded = _elu(genc * (xc * (1.0 / (sd + 1e-6))) + benc)
    enc16 = encoded.astype(BF16)
    enc_virt = _elu(benc)
    ev16 = enc_virt.astype(BF16)

    Wd_all = jnp.concatenate([Wdec_ref[0], Wdec_ref[1]],
                             axis=1).astype(BF16)
    Hd_all = jnp.dot(enc16, Wd_all, preferred_element_type=F32)
    Hv_all = jnp.dot(ev16, Wd_all, preferred_element_type=F32)
    Hdt_all = _t(Hd_all)
    Hvt_all = _t(Hv_all)

    rows = []
    for b in range(B):
        mcol = masksT[:, b:b + 1]
        dec = jnp.zeros((1, DEC), dtype=F32)
        for h in range(H):
            Hd = Hd_all[b * L:(b + 1) * L, h * DEC:(h + 1) * DEC]
            Hdt = Hdt_all[b * L:(b + 1) * L, h * DEC:(h + 1) * DEC]
            Hv = Hv_all[:, h * DEC:(h + 1) * DEC]
            Hvt = Hvt_all[:, h * DEC:(h + 1) * DEC]
            a1d = _t(adec_ref[h:h + 1, :DEC])
            a2d = _t(adec_ref[h:h + 1, DEC:])
            s1v = jnp.sum(Hvt * a1d)
            s2 = jnp.sum(Hdt * a2d, axis=1, keepdims=True)
            s2v = jnp.sum(Hvt * a2d)
            e = jnp.exp(-_lrelu(s1v + s2)) * mcol
            ev = jnp.exp(-_lrelu(s1v + s2v))
            den = jnp.sum(e) + ev + 1e-10
            wsum = jnp.sum(e * Hd, axis=0, keepdims=True) + ev * Hv
            dec = dec + 0.5 * wsum / den

        mu = jnp.mean(dec, axis=1, keepdims=True)
        xc = dec - mu
        sd = jnp.sqrt(jnp.sum(xc * xc, axis=1, keepdims=True) / (DEC - 1))
        lnd = gdec_ref[...] * xc / (sd + 1e-6) + bdec_ref[...]
        r = jnp.maximum(lnd, 0.0)
        rows.append(jnp.dot(r.astype(BF16), Vw_ref[...].astype(BF16),
                            preferred_element_type=F32) + Vb_ref[...])

    stacked = jnp.concatenate(rows, axis=0)
    h1 = jnp.maximum(
        jnp.dot(stacked.astype(BF16), w1_ref[...].astype(BF16),
                preferred_element_type=F32) + b1_ref[...], 0.0)
    pred = jnp.dot(h1.astype(BF16), w2_ref[...].astype(BF16),
                   preferred_element_type=F32) + b2_ref[...]
    out_ref[...] = pred


def kernel(data, params):
    args = (
        data,
        params["embed"],
        params["W_enc"],
        params["a_enc"],
        params["W_dec"],
        params["a_dec"],
        params["g_enc"].reshape(1, 2 * ENC),
        params["b_enc"].reshape(1, 2 * ENC),
        params["g_dec"].reshape(1, DEC),
        params["b_dec"].reshape(1, DEC),
        params["V_w"],
        params["V_b"].reshape(1, DEC),
        params["out_w1"],
        params["out_b1"].reshape(1, DEC),
        params["out_w2"],
        params["out_b2"].reshape(1, 1),
    )
    out = pl.pallas_call(
        _vgnn_kernel,
        out_shape=jax.ShapeDtypeStruct((B, 1), F32),
    )(*args)
    return (out, jnp.asarray(0.0, dtype=F32))


# speedup vs baseline: 5147.8923x; 1.0366x over previous
import jax
import jax.numpy as jnp
from jax.experimental import pallas as pl

L = 512
ENC = 256
DEC = 256
H = 2
B = 8
ALPHA = 0.2
F32 = jnp.float32
BF16 = jnp.bfloat16
HIGH = jax.lax.Precision.HIGHEST


def _lrelu(x):
    return jnp.maximum(x, ALPHA * x)


def _elu(x):
    return jnp.maximum(x, jnp.exp(jnp.minimum(x, 0.0)) - 1.0)


def _t(x):
    return x.astype(BF16).astype(F32)


def _split(a):
    ahi = a.astype(BF16)
    alo = (a - ahi.astype(F32)).astype(BF16)
    return ahi, alo


def _vgnn_kernel(data_ref, embed_ref, Wenc_ref, aenc_ref,
                 Wdec_ref, adec_ref, genc_ref, benc_ref, gdec_ref, bdec_ref,
                 Vw_ref, Vb_ref, w1_ref, b1_ref, w2_ref, b2_ref, out_ref):
    embed16 = embed_ref[:L, :].astype(BF16)
    masksT = data_ref[...].T

    Hsplit = []
    Ps = []
    for h in range(H):
        Hh = jnp.dot(embed16, Wenc_ref[h].astype(BF16),
                     preferred_element_type=F32)
        Hhi, Hlo = _split(Hh)
        a1c = aenc_ref[h:h + 1, :ENC].reshape(ENC, 1)
        a2c = aenc_ref[h:h + 1, ENC:].reshape(ENC, 1)
        amat = jnp.concatenate([a1c, a2c], axis=1).astype(BF16)
        s12 = jnp.dot(Hhi, amat, preferred_element_type=F32)
        s1 = s12[:, 0:1]
        s2 = s12[:, 1:2]
        x = s1 + s2.reshape(1, L)
        Ps.append(jnp.exp(-_lrelu(x)))
        Hsplit.append((Hhi, Hlo))

    masks = masksT
    totals = jnp.sum(masks, axis=0, keepdims=True)
    empty = (totals == 0).astype(F32)
    iota = jax.lax.broadcasted_iota(jnp.int32, (L, 1), 0)
    oh0 = (iota == 0).astype(F32)
    me_all = oh0 * empty + masks * (1.0 - empty)

    me16 = me_all.astype(BF16)
    zeros8 = jnp.zeros((L, B), dtype=BF16)
    Rs = []
    for h in range(H):
        Phi, Plo = _split(Ps[h])
        Hhi, Hlo = Hsplit[h]
        Xhi = jnp.concatenate(
            [Hhi * me16[:, b:b + 1] for b in range(B)] + [me16],
            axis=1)
        Xlo = jnp.concatenate(
            [Hlo * me16[:, b:b + 1] for b in range(B)] + [zeros8],
            axis=1)
        a2m = jnp.concatenate([Phi, Phi, Plo], axis=1)
        x2m = jnp.concatenate([Xhi, Xlo, Xhi], axis=0)
        Rs.append(jnp.dot(a2m, x2m, preferred_element_type=F32))

    blocks = []
    for b in range(B):
        me = me_all[:, b:b + 1]
        parts = []
        for h in range(H):
            numer = Rs[h][:, b * ENC:(b + 1) * ENC]
            denom = Rs[h][:, B * ENC + b:B * ENC + b + 1]
            w = me / (me * denom + 1e-10)
            parts.append(_elu(numer * w))
        blocks.append(jnp.concatenate(parts, axis=1))
    enc_raw = jnp.concatenate(blocks, axis=0)

    genc = genc_ref[...]
    benc = benc_ref[...]
    mu = jnp.mean(enc_raw, axis=1, keepdims=True)
    xc = enc_raw - mu
    sd = jnp.sqrt(jnp.sum(xc * xc, axis=1, keepdims=True) / (2 * ENC - 1))
    encoded = _elu(genc * (xc * (1.0 / (sd + 1e-6))) + benc)
    enc16 = encoded.astype(BF16)
    enc_virt = _elu(benc)
    ev16 = enc_virt.astype(BF16)

    Wd_all = jnp.concatenate([Wdec_ref[0], Wdec_ref[1]],
                             axis=1).astype(BF16)
    Hd_all = jnp.dot(enc16, Wd_all, preferred_element_type=F32)
    Hv_all = jnp.dot(ev16, Wd_all, preferred_element_type=F32)

    zc = jnp.zeros((DEC, 2), dtype=F32)
    atop = jnp.concatenate(
        [adec_ref[0:1, :DEC].reshape(DEC, 1),
         adec_ref[0:1, DEC:].reshape(DEC, 1), zc], axis=1)
    abot = jnp.concatenate(
        [zc, adec_ref[1:2, :DEC].reshape(DEC, 1),
         adec_ref[1:2, DEC:].reshape(DEC, 1)], axis=1)
    amat_dec = jnp.concatenate([atop, abot], axis=0).astype(BF16)
    S_all = jnp.dot(Hd_all.astype(BF16), amat_dec,
                    preferred_element_type=F32)
    Sv = jnp.dot(Hv_all.astype(BF16), amat_dec,
                 preferred_element_type=F32)

    rows = []
    for b in range(B):
        mcol = masksT[:, b:b + 1]
        dec = jnp.zeros((1, DEC), dtype=F32)
        for h in range(H):
            Hd = Hd_all[b * L:(b + 1) * L, h * DEC:(h + 1) * DEC]
            Hv = Hv_all[:, h * DEC:(h + 1) * DEC]
            s1v = Sv[0:1, 2 * h:2 * h + 1]
            s2 = S_all[b * L:(b + 1) * L, 2 * h + 1:2 * h + 2]
            s2v = Sv[0:1, 2 * h + 1:2 * h + 2]
            e = jnp.exp(-_lrelu(s1v + s2)) * mcol
            ev = jnp.exp(-_lrelu(s1v + s2v))
            den = jnp.sum(e) + jnp.sum(ev) + 1e-10
            wsum = jnp.sum(e * Hd, axis=0, keepdims=True) + ev * Hv
            dec = dec + 0.5 * wsum / den

        mu = jnp.mean(dec, axis=1, keepdims=True)
        xc = dec - mu
        sd = jnp.sqrt(jnp.sum(xc * xc, axis=1, keepdims=True) / (DEC - 1))
        lnd = gdec_ref[...] * xc / (sd + 1e-6) + bdec_ref[...]
        r = jnp.maximum(lnd, 0.0)
        rows.append(jnp.dot(r.astype(BF16), Vw_ref[...].astype(BF16),
                            preferred_element_type=F32) + Vb_ref[...])

    stacked = jnp.concatenate(rows, axis=0)
    h1 = jnp.maximum(
        jnp.dot(stacked.astype(BF16), w1_ref[...].astype(BF16),
                preferred_element_type=F32) + b1_ref[...], 0.0)
    pred = jnp.dot(h1.astype(BF16), w2_ref[...].astype(BF16),
                   preferred_element_type=F32) + b2_ref[...]
    out_ref[...] = pred


def kernel(data, params):
    args = (
        data,
        params["embed"],
        params["W_enc"],
        params["a_enc"],
        params["W_dec"],
        params["a_dec"],
        params["g_enc"].reshape(1, 2 * ENC),
        params["b_enc"].reshape(1, 2 * ENC),
        params["g_dec"].reshape(1, DEC),
        params["b_dec"].reshape(1, DEC),
        params["V_w"],
        params["V_b"].reshape(1, DEC),
        params["out_w1"],
        params["out_b1"].reshape(1, DEC),
        params["out_w2"],
        params["out_b2"].reshape(1, 1),
    )
    out = pl.pallas_call(
        _vgnn_kernel,
        out_shape=jax.ShapeDtypeStruct((B, 1), F32),
    )(*args)
    return (out, jnp.asarray(0.0, dtype=F32))
